# baseline (speedup 1.0000x reference)
import functools

import jax
import jax.numpy as jnp
from jax import lax
from jax.experimental import pallas as pl
from jax.experimental.pallas import tpu as pltpu
from jax.experimental.pallas import tpu_sc as plsc

N = 10000
E = 320000
D_IN = 128
D_H = 256
D_OUT = 128

NC = 2
NS = 16
CHUNK = 128
S0 = 80
S1 = 80
TOT = NS * (S0 + S1)
E_PAD = TOT * CHUNK
NACC = 10240
RPT = NACC // NS
ZR = 128
NBUF = 2
NIDX = 4
BF_ZR = 64


def _seg_sum_body(compute_deg, x_hbm, src_hbm, dst_hbm, *rest):
    if compute_deg:
        (acc_out, deg_out, acc_sh, deg_sh, ones_v, degc_v) = rest[:6]
        rest = rest[6:]
    else:
        acc_out, acc_sh = rest[:2]
        rest = rest[2:]
    rows = list(rest[0:NBUF])
    srcv = list(rest[NBUF:NBUF + NIDX])
    dstv = list(rest[NBUF + NIDX:NBUF + 2 * NIDX])
    rest = rest[NBUF + 2 * NIDX:]
    gsem = list(rest[0:NBUF])
    ssem = list(rest[NBUF:2 * NBUF])
    isem = list(rest[2 * NBUF:2 * NBUF + NIDX])
    dsem = list(rest[2 * NBUF + NIDX:]) if compute_deg else [None] * NBUF

    import contextlib
    scope = jax.named_scope

    c = lax.axis_index("c")
    s = lax.axis_index("s")
    base = s * RPT
    cbase = jnp.where(c == 0, s * S0, NS * S0 + s * S1)
    T = jnp.where(c == 0, S0, S1)
    G = jnp.where(c == 0, (S0 - 4) // 4, (S1 - 4) // 4)

    ctx_zero = scope("sc_zero_init")
    ctx_zero.__enter__()

    def zrow(i, carry):
        for j in range(8):
            rows[0][i, pl.ds(j * 16, 16)] = jnp.zeros((16,), jnp.float32)
        return carry
    lax.fori_loop(0, ZR, zrow, 0)

    for k in range(RPT // ZR):
        pltpu.sync_copy(rows[0], acc_sh.at[pl.ds(base + k * ZR, ZR)])

    if compute_deg:
        def orow(j, carry):
            ones_v[pl.ds(j * 16, 16)] = jnp.ones((16,), jnp.float32)
            return carry
        lax.fori_loop(0, CHUNK // 16, orow, 0)

        def drow(j, carry):
            degc_v[pl.ds(j * 16, 16)] = jnp.zeros((16,), jnp.float32)
            return carry
        lax.fori_loop(0, RPT // 16, drow, 0)
        pltpu.sync_copy(degc_v, deg_sh.at[pl.ds(base, RPT)])

    ctx_zero.__exit__(None, None, None)
    with scope("sc_barrier1"):
        plsc.subcore_barrier()
    ctx_edges = scope("sc_edges")
    ctx_edges.__enter__()

    def idxload(cid, q):
        pltpu.async_copy(src_hbm.at[cid], srcv[q], isem[q])
        pltpu.async_copy(dst_hbm.at[cid], dstv[q], isem[q])

    def wait_idx(q):
        pltpu.make_async_copy(src_hbm.at[0], srcv[q], isem[q]).wait()
        pltpu.make_async_copy(dst_hbm.at[0], dstv[q], isem[q]).wait()

    def gather(q, b):
        pltpu.async_copy(x_hbm.at[srcv[q]], rows[b], gsem[b])

    def wait_g(b):
        pltpu.make_async_copy(x_hbm.at[srcv[0]], rows[b], gsem[b]).wait()

    def scatters(q, b):
        pltpu.async_copy(rows[b], acc_sh.at[dstv[q]], ssem[b], add=True)
        if compute_deg:
            pltpu.async_copy(ones_v, deg_sh.at[dstv[q]], dsem[b], add=True)

    def wait_sc(b):
        pltpu.make_async_copy(rows[b], acc_sh.at[dstv[0]], ssem[b]).wait()
        if compute_deg:
            pltpu.make_async_copy(ones_v, deg_sh.at[dstv[0]],
                                  dsem[b]).wait()

    for q in range(NIDX - 1):
        idxload(cbase + q, q)
    wait_idx(0)
    gather(0, 0)

    wait_g(0)
    scatters(0, 0)
    wait_idx(1)
    gather(1, 1)
    idxload(cbase + 3, 3)
    wait_g(1)
    scatters(1, 1)
    wait_idx(2)
    wait_sc(0)
    gather(2, 0)
    idxload(cbase + 4, 0)

    def group(g, carry):
        jg = 2 + 4 * g
        for k in range(4):
            b = k % 2
            wait_g(b)
            scatters((2 + k) % 4, b)
            wait_idx((3 + k) % 4)
            wait_sc((k + 1) % 2)
            gather((3 + k) % 4, (k + 1) % 2)
            idxload(cbase + jg + k + 3, (1 + k) % 4)
        return carry
    lax.fori_loop(0, G, group, 0)

    wait_g(0)
    scatters(2, 0)
    wait_idx(3)
    wait_sc(1)
    gather(3, 1)
    wait_g(1)
    scatters(3, 1)

    wait_sc(0)
    wait_sc(1)
    wait_idx(0)

    ctx_edges.__exit__(None, None, None)
    with scope("sc_barrier2"):
        plsc.subcore_barrier()

    with scope("sc_readout"):
        pltpu.sync_copy(acc_sh.at[pl.ds(base, RPT)],
                        acc_out.at[c, pl.ds(base, RPT)])
        if compute_deg:
            pltpu.sync_copy(deg_sh.at[pl.ds(base, RPT)],
                            deg_out.at[c, pl.ds(base, RPT)])


def _make_seg_sum(compute_deg):
    mesh = plsc.VectorSubcoreMesh(core_axis_name="c", subcore_axis_name="s")
    out_type = [jax.ShapeDtypeStruct((NC, NACC, D_IN), jnp.float32)]
    scratch = [
        pltpu.VMEM_SHARED((NACC, D_IN), jnp.float32),
    ]
    if compute_deg:
        out_type.append(jax.ShapeDtypeStruct((NC, NACC), jnp.float32))
        scratch.append(pltpu.VMEM_SHARED((NACC,), jnp.float32))
        scratch += [
            pltpu.VMEM((CHUNK,), jnp.float32),
            pltpu.VMEM((RPT,), jnp.float32),
        ]
    scratch += [pltpu.VMEM((CHUNK, D_IN), jnp.float32)] * NBUF
    scratch += [pltpu.VMEM((CHUNK,), jnp.int32)] * NIDX
    scratch += [pltpu.VMEM((CHUNK,), jnp.int32)] * NIDX
    nsem = 2 * NBUF + NIDX + (NBUF if compute_deg else 0)
    scratch += [pltpu.SemaphoreType.DMA] * nsem
    return pl.kernel(
        functools.partial(_seg_sum_body, compute_deg),
        out_type=out_type,
        mesh=mesh,
        scratch_types=scratch,
        compiler_params=pltpu.CompilerParams(needs_layout_passes=False),
    )


_seg_sum_deg = _make_seg_sum(True)
_seg_sum = _make_seg_sum(False)


RB = 1000


def _tc1_body(part_ref, deg_ref, x_ref, wl1_ref, wr1_ref, b1_ref,
              wl2_ref, wr2_ref, p_ref, r_ref):
    agg = (part_ref[0].astype(jnp.float32)
           + part_ref[1].astype(jnp.float32))
    d = jnp.maximum(deg_ref[0] + deg_ref[1], 1.0)
    agg = agg / d
    h = (jnp.dot(agg, wl1_ref[...], preferred_element_type=jnp.float32)
         + jnp.dot(x_ref[...], wr1_ref[...], preferred_element_type=jnp.float32)
         + b1_ref[...])
    h = jnp.maximum(h, 0.0)
    p_ref[...] = jnp.dot(h, wl2_ref[...], preferred_element_type=jnp.float32)
    r_ref[...] = jnp.dot(h, wr2_ref[...], preferred_element_type=jnp.float32)


def _tc2_body(part_ref, deg_ref, r_ref, b2_ref, out_ref):
    agg = (part_ref[0].astype(jnp.float32)
           + part_ref[1].astype(jnp.float32))
    d = jnp.maximum(deg_ref[0] + deg_ref[1], 1.0)
    out_ref[...] = agg / d + b2_ref[...] + r_ref[...]


def _tc1(part, deg, x, wl1, wr1, b1, wl2, wr2):
    grid = (N // RB,)
    return pl.pallas_call(
        _tc1_body,
        grid=grid,
        in_specs=[
            pl.BlockSpec((NC, RB, D_IN), lambda i: (0, i, 0)),
            pl.BlockSpec((NC, RB, 1), lambda i: (0, i, 0)),
            pl.BlockSpec((RB, D_IN), lambda i: (i, 0)),
            pl.BlockSpec((D_IN, D_H), lambda i: (0, 0)),
            pl.BlockSpec((D_IN, D_H), lambda i: (0, 0)),
            pl.BlockSpec((1, D_H), lambda i: (0, 0)),
            pl.BlockSpec((D_H, D_OUT), lambda i: (0, 0)),
            pl.BlockSpec((D_H, D_OUT), lambda i: (0, 0)),
        ],
        out_specs=[
            pl.BlockSpec((RB, D_OUT), lambda i: (i, 0)),
            pl.BlockSpec((RB, D_OUT), lambda i: (i, 0)),
        ],
        out_shape=[
            jax.ShapeDtypeStruct((N, D_OUT), jnp.float32),
            jax.ShapeDtypeStruct((N, D_OUT), jnp.float32),
        ],
    )(part, deg, x, wl1, wr1, b1, wl2, wr2)


def _tc2(part, deg, r, b2):
    grid = (N // RB,)
    return pl.pallas_call(
        _tc2_body,
        grid=grid,
        in_specs=[
            pl.BlockSpec((NC, RB, D_OUT), lambda i: (0, i, 0)),
            pl.BlockSpec((NC, RB, 1), lambda i: (0, i, 0)),
            pl.BlockSpec((RB, D_OUT), lambda i: (i, 0)),
            pl.BlockSpec((1, D_OUT), lambda i: (0, 0)),
        ],
        out_specs=pl.BlockSpec((RB, D_OUT), lambda i: (i, 0)),
        out_shape=jax.ShapeDtypeStruct((N, D_OUT), jnp.float32),
    )(part, deg, r, b2)


def kernel(x, edge_index, Wl1, Wr1, b1, Wl2, Wr2, b2):
    src = edge_index[0]
    dst = edge_index[1]
    pad = (TOT + 1) * CHUNK - E
    src2 = jnp.concatenate(
        [src, jnp.zeros((pad,), jnp.int32)]).reshape(TOT + 1, CHUNK)
    dummy = N + (jnp.arange(pad, dtype=jnp.int32) % (NACC - N))
    dst2 = jnp.concatenate([dst, dummy]).reshape(TOT + 1, CHUNK)

    part_x, deg = _seg_sum_deg(x, src2, dst2)
    deg3 = deg.reshape(NC, NACC, 1)
    p, r = _tc1(part_x, deg3, x, Wl1, Wr1, b1.reshape(1, D_H), Wl2, Wr2)
    part_p, = _seg_sum(p, src2, dst2)
    out = _tc2(part_p, deg3, r, b2.reshape(1, D_OUT))
    return out


# speedup vs baseline: 3.2320x; 3.2320x over previous
import functools

import jax
import jax.numpy as jnp
from jax import lax
from jax.experimental import pallas as pl
from jax.experimental.pallas import tpu as pltpu
from jax.experimental.pallas import tpu_sc as plsc

N = 10000
E = 320000
D_IN = 128
D_H = 256
D_OUT = 128

NC = 2
NS = 16
CHUNK = 128
S0 = 80
S1 = 80
TOT = NS * (S0 + S1)
E_PAD = TOT * CHUNK
NACC = 10240
RPT = NACC // NS
ZR = 128
NBUF = 2
NIDX = 4
BF_ZR = 64


def _seg_sum_body(compute_deg, x_hbm, src_hbm, dst_hbm, *rest):
    if compute_deg:
        (acc_out, deg_out, acc_sh, deg_sh, ones_v, degc_v) = rest[:6]
        rest = rest[6:]
    else:
        acc_out, acc_sh = rest[:2]
        rest = rest[2:]
    rows = list(rest[0:NBUF])
    srcv = list(rest[NBUF:NBUF + NIDX])
    dstv = list(rest[NBUF + NIDX:NBUF + 2 * NIDX])
    rest = rest[NBUF + 2 * NIDX:]
    gsem = list(rest[0:NBUF])
    ssem = list(rest[NBUF:2 * NBUF])
    isem = list(rest[2 * NBUF:2 * NBUF + NIDX])
    dsem = list(rest[2 * NBUF + NIDX:]) if compute_deg else [None] * NBUF

    import contextlib
    scope = jax.named_scope

    c = lax.axis_index("c")
    s = lax.axis_index("s")
    base = s * RPT
    cbase = jnp.where(c == 0, s * S0, NS * S0 + s * S1)
    T = jnp.where(c == 0, S0, S1)
    G = jnp.where(c == 0, (S0 - 4) // 4, (S1 - 4) // 4)

    ctx_zero = scope("sc_zero_init")
    ctx_zero.__enter__()

    def zrow(i, carry):
        for j in range(8):
            rows[0][i, pl.ds(j * 16, 16)] = jnp.zeros((16,), jnp.float32)
        return carry
    lax.fori_loop(0, ZR, zrow, 0)

    for k in range(RPT // ZR):
        pltpu.sync_copy(rows[0], acc_sh.at[pl.ds(base + k * ZR, ZR)])

    if compute_deg:
        def orow(j, carry):
            ones_v[pl.ds(j * 16, 16)] = jnp.ones((16,), jnp.float32)
            return carry
        lax.fori_loop(0, CHUNK // 16, orow, 0)

        def drow(j, carry):
            degc_v[pl.ds(j * 16, 16)] = jnp.zeros((16,), jnp.float32)
            return carry
        lax.fori_loop(0, RPT // 16, drow, 0)
        pltpu.sync_copy(degc_v, deg_sh.at[pl.ds(base, RPT)])

    ctx_zero.__exit__(None, None, None)
    with scope("sc_barrier1"):
        plsc.subcore_barrier()
    ctx_edges = scope("sc_edges")
    ctx_edges.__enter__()

    def idxload(cid, q):
        pltpu.async_copy(src_hbm.at[cid], srcv[q], isem[q])
        pltpu.async_copy(dst_hbm.at[cid], dstv[q], isem[q])

    def wait_idx(q):
        pltpu.make_async_copy(src_hbm.at[0], srcv[q], isem[q]).wait()
        pltpu.make_async_copy(dst_hbm.at[0], dstv[q], isem[q]).wait()

    def gather(q, b):
        pltpu.async_copy(x_hbm.at[srcv[q]], rows[b], gsem[b])

    def wait_g(b):
        pltpu.make_async_copy(x_hbm.at[srcv[0]], rows[b], gsem[b]).wait()

    def scatters(q, b):
        pltpu.async_copy(rows[b], acc_sh.at[dstv[q]], ssem[b], add=True)
        if compute_deg:
            pltpu.async_copy(ones_v, deg_sh.at[dstv[q]], dsem[b], add=True)

    def wait_sc(b):
        pltpu.make_async_copy(rows[b], acc_sh.at[dstv[0]], ssem[b]).wait()
        if compute_deg:
            pltpu.make_async_copy(ones_v, deg_sh.at[dstv[0]],
                                  dsem[b]).wait()

    for q in range(NIDX - 1):
        idxload(cbase + q, q)
    wait_idx(0)
    gather(0, 0)

    wait_g(0)
    scatters(0, 0)
    wait_idx(1)
    gather(1, 1)
    idxload(cbase + 3, 3)
    wait_g(1)
    scatters(1, 1)
    wait_idx(2)
    wait_sc(0)
    gather(2, 0)
    idxload(cbase + 4, 0)

    def group(g, carry):
        jg = 2 + 4 * g
        for k in range(4):
            b = k % 2
            wait_g(b)
            scatters((2 + k) % 4, b)
            wait_idx((3 + k) % 4)
            wait_sc((k + 1) % 2)
            gather((3 + k) % 4, (k + 1) % 2)
            idxload(cbase + jg + k + 3, (1 + k) % 4)
        return carry
    lax.fori_loop(0, G, group, 0)

    wait_g(0)
    scatters(2, 0)
    wait_idx(3)
    wait_sc(1)
    gather(3, 1)
    wait_g(1)
    scatters(3, 1)

    wait_sc(0)
    wait_sc(1)
    wait_idx(0)

    ctx_edges.__exit__(None, None, None)
    with scope("sc_barrier2"):
        plsc.subcore_barrier()

    with scope("sc_readout"):
        pltpu.sync_copy(acc_sh.at[pl.ds(base, RPT)],
                        acc_out.at[c, pl.ds(base, RPT)])
        if compute_deg:
            pltpu.sync_copy(deg_sh.at[pl.ds(base, RPT)],
                            deg_out.at[c, pl.ds(base, RPT)])


def _make_seg_sum(compute_deg):
    mesh = plsc.VectorSubcoreMesh(core_axis_name="c", subcore_axis_name="s")
    out_type = [jax.ShapeDtypeStruct((NC, NACC, D_IN), jnp.float32)]
    scratch = [
        pltpu.VMEM_SHARED((NACC, D_IN), jnp.float32),
    ]
    if compute_deg:
        out_type.append(jax.ShapeDtypeStruct((NC, NACC), jnp.float32))
        scratch.append(pltpu.VMEM_SHARED((NACC,), jnp.float32))
        scratch += [
            pltpu.VMEM((CHUNK,), jnp.float32),
            pltpu.VMEM((RPT,), jnp.float32),
        ]
    scratch += [pltpu.VMEM((CHUNK, D_IN), jnp.float32)] * NBUF
    scratch += [pltpu.VMEM((CHUNK,), jnp.int32)] * NIDX
    scratch += [pltpu.VMEM((CHUNK,), jnp.int32)] * NIDX
    nsem = 2 * NBUF + NIDX + (NBUF if compute_deg else 0)
    scratch += [pltpu.SemaphoreType.DMA] * nsem
    return pl.kernel(
        functools.partial(_seg_sum_body, compute_deg),
        out_type=out_type,
        mesh=mesh,
        scratch_types=scratch,
        compiler_params=pltpu.CompilerParams(needs_layout_passes=False),
    )


_seg_sum_deg = _make_seg_sum(True)
_seg_sum = _make_seg_sum(False)


RB = 1000


def _tc1_body(part_ref, deg_ref, x_ref, wl1_ref, wr1_ref, b1_ref,
              wl2_ref, wr2_ref, p_ref, r_ref):
    agg = (part_ref[0].astype(jnp.float32)
           + part_ref[1].astype(jnp.float32))
    d = jnp.maximum(deg_ref[0] + deg_ref[1], 1.0)
    agg = agg / d
    h = (jnp.dot(agg, wl1_ref[...], preferred_element_type=jnp.float32)
         + jnp.dot(x_ref[...], wr1_ref[...], preferred_element_type=jnp.float32)
         + b1_ref[...])
    h = jnp.maximum(h, 0.0)
    p_ref[...] = jnp.dot(h, wl2_ref[...], preferred_element_type=jnp.float32)
    r_ref[...] = jnp.dot(h, wr2_ref[...], preferred_element_type=jnp.float32)


def _tc2_body(part_ref, deg_ref, r_ref, b2_ref, out_ref):
    agg = (part_ref[0].astype(jnp.float32)
           + part_ref[1].astype(jnp.float32))
    d = jnp.maximum(deg_ref[0] + deg_ref[1], 1.0)
    out_ref[...] = agg / d + b2_ref[...] + r_ref[...]


def _tc1(part, deg, x, wl1, wr1, b1, wl2, wr2):
    grid = (N // RB,)
    return pl.pallas_call(
        _tc1_body,
        grid=grid,
        in_specs=[
            pl.BlockSpec((NC, RB, D_IN), lambda i: (0, i, 0)),
            pl.BlockSpec((NC, RB, 1), lambda i: (0, i, 0)),
            pl.BlockSpec((RB, D_IN), lambda i: (i, 0)),
            pl.BlockSpec((D_IN, D_H), lambda i: (0, 0)),
            pl.BlockSpec((D_IN, D_H), lambda i: (0, 0)),
            pl.BlockSpec((1, D_H), lambda i: (0, 0)),
            pl.BlockSpec((D_H, D_OUT), lambda i: (0, 0)),
            pl.BlockSpec((D_H, D_OUT), lambda i: (0, 0)),
        ],
        out_specs=[
            pl.BlockSpec((RB, D_OUT), lambda i: (i, 0)),
            pl.BlockSpec((RB, D_OUT), lambda i: (i, 0)),
        ],
        out_shape=[
            jax.ShapeDtypeStruct((N, D_OUT), jnp.float32),
            jax.ShapeDtypeStruct((N, D_OUT), jnp.float32),
        ],
    )(part, deg, x, wl1, wr1, b1, wl2, wr2)


def _tc2(part, deg, r, b2):
    grid = (N // RB,)
    return pl.pallas_call(
        _tc2_body,
        grid=grid,
        in_specs=[
            pl.BlockSpec((NC, RB, D_OUT), lambda i: (0, i, 0)),
            pl.BlockSpec((NC, RB, 1), lambda i: (0, i, 0)),
            pl.BlockSpec((RB, D_OUT), lambda i: (i, 0)),
            pl.BlockSpec((1, D_OUT), lambda i: (0, 0)),
        ],
        out_specs=pl.BlockSpec((RB, D_OUT), lambda i: (i, 0)),
        out_shape=jax.ShapeDtypeStruct((N, D_OUT), jnp.float32),
    )(part, deg, r, b2)


def kernel(x, edge_index, Wl1, Wr1, b1, Wl2, Wr2, b2):
    src = edge_index[0]
    dst = edge_index[1]
    pad = (TOT + 1) * CHUNK - E
    pad_iota = jnp.arange(pad, dtype=jnp.int32)
    src2 = jnp.concatenate(
        [src, pad_iota % N]).reshape(TOT + 1, CHUNK)
    dst2 = jnp.concatenate(
        [dst, N + pad_iota % (NACC - N)]).reshape(TOT + 1, CHUNK)

    part_x, deg = _seg_sum_deg(x, src2, dst2)
    deg3 = deg.reshape(NC, NACC, 1)
    p, r = _tc1(part_x, deg3, x, Wl1, Wr1, b1.reshape(1, D_H), Wl2, Wr2)
    part_p, = _seg_sum(p, src2, dst2)
    out = _tc2(part_p, deg3, r, b2.reshape(1, D_OUT))
    return out
